# baseline (speedup 1.0000x reference)
import functools

import jax
import jax.numpy as jnp
from jax import lax
from jax.experimental import pallas as pl
from jax.experimental.pallas import tpu as pltpu
from jax.experimental.pallas import tpu_sc as plsc

K0, K1, K2 = 3, 3, 3
KPROD = K0 * K1 * K2
NL = 16
GC = 16
NBR_CAP = 4112
ACC_ROWS = 40

_TAKE_DNUMS = lax.GatherDimensionNumbers(
    offset_dims=(), collapsed_slice_dims=(0,), start_index_map=(0,))


def _take(v, idx):
    return lax.gather(v, idx[:, None], _TAKE_DNUMS, (1,),
                      mode=lax.GatherScatterMode.PROMISE_IN_BOUNDS)


def _sc_body(px_h, py_h, pz_h, qx_h, qy_h, qz_h, r_h, fw_h, wsum_h,
             pxv, pyv, pzv, featv, qxv, qyv, qzv, rv, nbrv, accv, outv,
             spx, spy, spz, sidx, cidv, cellst, cursor, sem):
    n = px_h.shape[0]
    m_total = wsum_h.shape[0] // (KPROD * 32)
    cid = lax.axis_index("c")
    sid = lax.axis_index("s")
    wid = sid * 2 + cid
    m_per = m_total // 32
    base = wid * m_per

    pltpu.sync_copy(px_h, pxv)
    pltpu.sync_copy(py_h, pyv)
    pltpu.sync_copy(pz_h, pzv)
    pltpu.sync_copy(fw_h, featv)
    pltpu.sync_copy(qx_h.at[pl.ds(base, m_per)], qxv)
    pltpu.sync_copy(qy_h.at[pl.ds(base, m_per)], qyv)
    pltpu.sync_copy(qz_h.at[pl.ds(base, m_per)], qzv)
    pltpu.sync_copy(r_h.at[pl.ds(base, m_per)], rv)

    iota = lax.iota(jnp.int32, NL)
    zi = iota * 0
    zf = zi.astype(jnp.float32)

    def splat(v, j):
        return _take(v, zi + j)

    fgc = float(GC)

    def cbody(t, _):
        off = t * NL
        yc = jnp.clip((pyv[pl.ds(off, NL)] * fgc).astype(jnp.int32),
                      0, GC - 1)
        zc = jnp.clip((pzv[pl.ds(off, NL)] * fgc).astype(jnp.int32),
                      0, GC - 1)
        cidv[pl.ds(off, NL)] = zc * GC + yc
        return 0

    lax.fori_loop(0, n // NL, cbody, 0)

    for i in range(17):
        cursor[pl.ds(i * NL, NL)] = zi

    def hbody(t, _):
        c = cidv[pl.ds(t * NL, NL)]
        dup, last = plsc.scan_count(c)
        plsc.addupdate_scatter(cursor, [c], dup, mask=last)
        return 0

    lax.fori_loop(0, n // NL, hbody, 0)

    carry = zi
    for i in range(GC * GC // NL):
        v = cursor[pl.ds(i * NL, NL)]
        cs = plsc.cumsum(v)
        cellst[pl.ds(i * NL, NL)] = carry + (cs - v)
        carry = carry + splat(cs, NL - 1)
    cellst[pl.ds(GC * GC, NL)] = carry

    for i in range(17):
        cursor[pl.ds(i * NL, NL)] = cellst[pl.ds(i * NL, NL)]

    def sbody(t, _):
        off = t * NL
        c = cidv[pl.ds(off, NL)]
        dup, last = plsc.scan_count(c)
        cur = plsc.load_gather(cursor, [c])
        pos = cur + dup - 1
        plsc.store_scatter(sidx, [pos], iota + off)
        plsc.store_scatter(spx, [pos], pxv[pl.ds(off, NL)])
        plsc.store_scatter(spy, [pos], pyv[pl.ds(off, NL)])
        plsc.store_scatter(spz, [pos], pzv[pl.ds(off, NL)])
        plsc.addupdate_scatter(cursor, [c], dup, mask=last)
        return 0

    lax.fori_loop(0, n // NL, sbody, 0)

    GRP = 8

    def mbody(m, carry_):
        gb = (m // NL) * NL
        qx16 = qxv[pl.ds(gb, NL)]
        qy16 = qyv[pl.ds(gb, NL)]
        qz16 = qzv[pl.ds(gb, NL)]
        r16 = rv[pl.ds(gb, NL)]
        qxs = splat(qx16, m - gb)
        qys = splat(qy16, m - gb)
        qzs = splat(qz16, m - gb)
        rs = splat(r16, m - gb)
        r2s = rs * rs
        inv_rs = 1.0 / rs

        y0v = jnp.clip(((qys - rs) * fgc).astype(jnp.int32), 0, GC - 1)
        y1v = jnp.clip(((qys + rs) * fgc).astype(jnp.int32), 0, GC - 1)
        z0v = jnp.clip(((qzs - rs) * fgc).astype(jnp.int32), 0, GC - 1)
        z1v = jnp.clip(((qzs + rs) * fgc).astype(jnp.int32), 0, GC - 1)
        zpk = jnp.max(z0v * 256 + z1v)
        z0 = zpk >> 8
        z1 = zpk & 255

        def zbody(zc, cv_in):
            zb = zi + zc * GC
            st = plsc.load_gather(cellst, [zb + y0v])
            en = plsc.load_gather(cellst, [zb + y1v + 1])
            nch = jnp.max(en - st)

            def tbody(t, cv):
                idxs = st + iota + t * NL
                ok = idxs < en
                idc = jnp.where(ok, idxs, 0)
                gx = plsc.load_gather(spx, [idc])
                gy = plsc.load_gather(spy, [idc])
                gz = plsc.load_gather(spz, [idc])
                oid = plsc.load_gather(sidx, [idc])
                dx = gx - qxs
                dy = gy - qys
                dz = gz - qzs
                d2 = dx * dx + dy * dy + dz * dz
                msk = (d2 <= r2s) & ok
                hits = plsc.all_reduce_population_count(msk)
                pos = cv + plsc.cumsum(msk.astype(jnp.int32)) - 1
                plsc.store_scatter(nbrv, [pos], oid, mask=msk)
                return cv + hits

            return plsc.parallel_loop(
                0, (nch + NL - 1) // NL, carry=cv_in)(tbody)

        cnt_vec = lax.fori_loop(z0, z1 + 1, zbody, zi)
        cnt = jnp.max(cnt_vec)
        plsc.store_scatter(nbrv, [cnt_vec + iota], zi)

        for i in range(2 * KPROD):
            accv[pl.ds(i * NL, NL)] = zf

        def bbody(jb):
            j16 = jb * NL
            idx = nbrv[pl.ds(j16, NL)]
            gx = plsc.load_gather(pxv, [idx])
            gy = plsc.load_gather(pyv, [idx])
            gz = plsc.load_gather(pzv, [idx])
            dx = gx - qxs
            dy = gy - qys
            dz = gz - qzs
            d2 = jnp.maximum(dx * dx + dy * dy + dz * dz, 1e-20)
            ib = plsc.bitcast(d2, jnp.int32)
            y = plsc.bitcast(jnp.int32(0x5F3759DF) - (ib >> 1), jnp.float32)
            y = y * (1.5 - 0.5 * d2 * y * y)
            y = y * (1.5 - 0.5 * d2 * y * y)
            sq = d2 * y
            relx = dx * inv_rs
            rely = dy * inv_rs
            relz = dz * inv_rs
            norm = sq * inv_rs
            ma = jnp.maximum(jnp.maximum(jnp.abs(relx), jnp.abs(rely)),
                             jnp.abs(relz))
            scale = jnp.where(ma > 1e-8, norm / jnp.maximum(ma, 1e-8), 0.0)
            t0 = jnp.clip(relx * scale + 1.0, 0.0, 2.0)
            t1 = jnp.clip(rely * scale + 1.0, 0.0, 2.0)
            t2 = jnp.clip(relz * scale + 1.0, 0.0, 2.0)
            lo0 = t0.astype(jnp.int32)
            lo1 = t1.astype(jnp.int32)
            lo2 = t2.astype(jnp.int32)
            f0 = t0 - lo0.astype(jnp.float32)
            f1 = t1 - lo1.astype(jnp.float32)
            f2 = t2 - lo2.astype(jnp.float32)
            kb32 = (lo0 * 9 + lo1 * 3 + lo2) * 32
            wb = idx * NL
            validf = ((iota + j16) < cnt_vec).astype(jnp.float32)

            for j in range(NL):
                jj = zi + j
                row = plsc.load_gather(featv, [_take(wb, jj) + iota])
                fa, fb = plsc.unpack(
                    plsc.bitcast(row, jnp.bfloat16),
                    format=plsc.PackFormat.INTERLEAVED)
                f0j = _take(f0, jj)
                f1j = _take(f1, jj)
                f2j = _take(f2, jj)
                aj = _take(validf, jj)
                addr = _take(kb32, jj) + iota
                g0 = aj - f0j * aj
                h0 = f0j * aj
                g1 = 1.0 - f1j
                g2 = 1.0 - f2j
                pgg = g1 * g2
                pfg = f1j * g2
                pgf = g1 * f2j
                pff = f1j * f2j
                for s0, w0 in ((0, g0), (1, h0)):
                    for (s1, s2), p12 in (((0, 0), pgg), ((0, 1), pgf),
                                          ((1, 0), pfg), ((1, 1), pff)):
                        w = w0 * p12
                        o = (s0 * 9 + s1 * 3 + s2) * 32
                        plsc.addupdate_scatter(accv, [addr + o], w * fa)
                        plsc.addupdate_scatter(accv, [addr + (o + 16)],
                                               w * fb)

        nb = (cnt + NL - 1) // NL
        plsc.parallel_loop(0, nb)(bbody)

        obase = (m % 2) * 864
        ov = outv.at[pl.ds(obase, 864)]

        @pl.when(m > 1)
        def _():
            pltpu.make_async_copy(
                ov, wsum_h.at[pl.ds((base + m - 2) * 864, 864)],
                sem).wait()

        inv_cnt = 1.0 / jnp.maximum(cnt_vec.astype(jnp.float32), 1.0)
        for i in range(KPROD * 2):
            ov[pl.ds(i * NL, NL)] = accv[pl.ds(i * NL, NL)] * inv_cnt
        pltpu.async_copy(ov, wsum_h.at[pl.ds((base + m) * 864, 864)], sem)
        return carry_

    lax.fori_loop(0, m_per, mbody, 0)
    for mm in (m_per - 2, m_per - 1):
        pltpu.make_async_copy(
            outv.at[pl.ds((mm % 2) * 864, 864)],
            wsum_h.at[pl.ds((base + mm) * 864, 864)], sem).wait()


def _mm_body(ws_ref, wr_ref, b_ref, o_ref):
    o_ref[...] = (jnp.dot(ws_ref[...], wr_ref[...],
                          preferred_element_type=jnp.float32)
                  + b_ref[0, :][None, :])


def kernel(inp_features, inp_positions, out_positions, extents, kernel, bias):
    n, cin = inp_features.shape
    m = out_positions.shape[0]
    cout = kernel.shape[-1]
    half = cin // 2

    px = inp_positions[:, 0].reshape(n)
    py = inp_positions[:, 1].reshape(n)
    pz = inp_positions[:, 2].reshape(n)
    qx = out_positions[:, 0].reshape(m)
    qy = out_positions[:, 1].reshape(m)
    qz = out_positions[:, 2].reshape(m)
    radii = (0.5 * extents).reshape(m)

    fb = inp_features.astype(jnp.bfloat16)
    lo = lax.bitcast_convert_type(fb[:, :half], jnp.uint16).astype(jnp.uint32)
    hi = lax.bitcast_convert_type(fb[:, half:], jnp.uint16).astype(jnp.uint32)
    featw = lax.bitcast_convert_type(lo | (hi << 16),
                                     jnp.int32).reshape(n * half)

    mesh = plsc.VectorSubcoreMesh(core_axis_name="c", subcore_axis_name="s")
    wsum = pl.kernel(
        _sc_body,
        out_type=jax.ShapeDtypeStruct((m * KPROD * cin,), jnp.float32),
        mesh=mesh,
        scratch_types=[
            pltpu.VMEM((n,), jnp.float32),
            pltpu.VMEM((n,), jnp.float32),
            pltpu.VMEM((n,), jnp.float32),
            pltpu.VMEM((n * half,), jnp.int32),
            pltpu.VMEM((m // 32,), jnp.float32),
            pltpu.VMEM((m // 32,), jnp.float32),
            pltpu.VMEM((m // 32,), jnp.float32),
            pltpu.VMEM((m // 32,), jnp.float32),
            pltpu.VMEM((NBR_CAP,), jnp.int32),
            pltpu.VMEM((ACC_ROWS * 32,), jnp.float32),
            pltpu.VMEM((2 * 8 * KPROD * 32,), jnp.float32),
            pltpu.VMEM((n,), jnp.float32),
            pltpu.VMEM((n,), jnp.float32),
            pltpu.VMEM((n,), jnp.float32),
            pltpu.VMEM((n,), jnp.int32),
            pltpu.VMEM((n,), jnp.int32),
            pltpu.VMEM((272,), jnp.int32),
            pltpu.VMEM((272,), jnp.int32),
            pltpu.SemaphoreType.DMA,
        ],
        compiler_params=pltpu.CompilerParams(needs_layout_passes=False),
    )(px, py, pz, qx, qy, qz, radii, featw)

    wr = kernel.reshape(KPROD * cin, cout)
    bias2 = bias.reshape(1, cout)
    out = pl.pallas_call(
        _mm_body,
        out_shape=jax.ShapeDtypeStruct((m, cout), jnp.float32),
    )(wsum.reshape(m, KPROD * cin), wr, bias2)
    return out


# speedup vs baseline: 1.1075x; 1.1075x over previous
import functools

import jax
import jax.numpy as jnp
from jax import lax
from jax.experimental import pallas as pl
from jax.experimental.pallas import tpu as pltpu
from jax.experimental.pallas import tpu_sc as plsc

K0, K1, K2 = 3, 3, 3
KPROD = K0 * K1 * K2
NL = 16
GC = 16
NBR_CAP = 4112
ACC_ROWS = 40

_TAKE_DNUMS = lax.GatherDimensionNumbers(
    offset_dims=(), collapsed_slice_dims=(0,), start_index_map=(0,))


def _take(v, idx):
    return lax.gather(v, idx[:, None], _TAKE_DNUMS, (1,),
                      mode=lax.GatherScatterMode.PROMISE_IN_BOUNDS)


def _sc_body(px_h, py_h, pz_h, qx_h, qy_h, qz_h, r_h, fw_h, wsum_h,
             pxv, pyv, pzv, featv, qxv, qyv, qzv, rv, nbrv, accv, outv,
             spx, spy, spz, sidx, cidv, cellst, cursor, sem):
    n = px_h.shape[0]
    m_total = wsum_h.shape[0] // (KPROD * 32)
    cid = lax.axis_index("c")
    sid = lax.axis_index("s")
    wid = sid * 2 + cid
    m_per = m_total // 32
    base = wid * m_per

    pltpu.sync_copy(px_h, pxv)
    pltpu.sync_copy(py_h, pyv)
    pltpu.sync_copy(pz_h, pzv)
    pltpu.sync_copy(fw_h, featv)
    pltpu.sync_copy(qx_h.at[pl.ds(base, m_per)], qxv)
    pltpu.sync_copy(qy_h.at[pl.ds(base, m_per)], qyv)
    pltpu.sync_copy(qz_h.at[pl.ds(base, m_per)], qzv)
    pltpu.sync_copy(r_h.at[pl.ds(base, m_per)], rv)

    iota = lax.iota(jnp.int32, NL)
    zi = iota * 0
    zf = zi.astype(jnp.float32)

    def splat(v, j):
        return _take(v, zi + j)

    fgc = float(GC)

    def cbody(t, _):
        off = t * NL
        yc = jnp.clip((pyv[pl.ds(off, NL)] * fgc).astype(jnp.int32),
                      0, GC - 1)
        zc = jnp.clip((pzv[pl.ds(off, NL)] * fgc).astype(jnp.int32),
                      0, GC - 1)
        cidv[pl.ds(off, NL)] = zc * GC + yc
        return 0

    lax.fori_loop(0, n // NL, cbody, 0)

    for i in range(17):
        cursor[pl.ds(i * NL, NL)] = zi

    def hbody(t, _):
        c = cidv[pl.ds(t * NL, NL)]
        dup, last = plsc.scan_count(c)
        plsc.addupdate_scatter(cursor, [c], dup, mask=last)
        return 0

    lax.fori_loop(0, n // NL, hbody, 0)

    carry = zi
    for i in range(GC * GC // NL):
        v = cursor[pl.ds(i * NL, NL)]
        cs = plsc.cumsum(v)
        cellst[pl.ds(i * NL, NL)] = carry + (cs - v)
        carry = carry + splat(cs, NL - 1)
    cellst[pl.ds(GC * GC, NL)] = carry

    for i in range(17):
        cursor[pl.ds(i * NL, NL)] = cellst[pl.ds(i * NL, NL)]

    def sbody(t, _):
        off = t * NL
        c = cidv[pl.ds(off, NL)]
        dup, last = plsc.scan_count(c)
        cur = plsc.load_gather(cursor, [c])
        pos = cur + dup - 1
        plsc.store_scatter(sidx, [pos], iota + off)
        plsc.store_scatter(spx, [pos], pxv[pl.ds(off, NL)])
        plsc.store_scatter(spy, [pos], pyv[pl.ds(off, NL)])
        plsc.store_scatter(spz, [pos], pzv[pl.ds(off, NL)])
        plsc.addupdate_scatter(cursor, [c], dup, mask=last)
        return 0

    lax.fori_loop(0, n // NL, sbody, 0)

    GRP = 8

    def mbody(m, carry_):
        gb = (m // NL) * NL
        qx16 = qxv[pl.ds(gb, NL)]
        qy16 = qyv[pl.ds(gb, NL)]
        qz16 = qzv[pl.ds(gb, NL)]
        r16 = rv[pl.ds(gb, NL)]
        qxs = splat(qx16, m - gb)
        qys = splat(qy16, m - gb)
        qzs = splat(qz16, m - gb)
        rs = splat(r16, m - gb)
        r2s = rs * rs
        inv_rs = 1.0 / rs

        y0v = jnp.clip(((qys - rs) * fgc).astype(jnp.int32), 0, GC - 1)
        y1v = jnp.clip(((qys + rs) * fgc).astype(jnp.int32), 0, GC - 1)
        z0v = jnp.clip(((qzs - rs) * fgc).astype(jnp.int32), 0, GC - 1)
        z1v = jnp.clip(((qzs + rs) * fgc).astype(jnp.int32), 0, GC - 1)
        zpk = jnp.max(z0v * 256 + z1v)
        z0 = zpk >> 8
        z1 = zpk & 255

        def zbody(zc, cv_in):
            zb = zi + zc * GC
            st = plsc.load_gather(cellst, [zb + y0v])
            en = plsc.load_gather(cellst, [zb + y1v + 1])
            nch = jnp.max(en - st)

            def tbody(t, cv):
                idxs = st + iota + t * NL
                ok = idxs < en
                idc = jnp.where(ok, idxs, 0)
                gx = plsc.load_gather(spx, [idc])
                gy = plsc.load_gather(spy, [idc])
                gz = plsc.load_gather(spz, [idc])
                oid = plsc.load_gather(sidx, [idc])
                dx = gx - qxs
                dy = gy - qys
                dz = gz - qzs
                d2 = dx * dx + dy * dy + dz * dz
                msk = (d2 <= r2s) & ok
                hits = plsc.all_reduce_population_count(msk)
                pos = cv + plsc.cumsum(msk.astype(jnp.int32)) - 1
                plsc.store_scatter(nbrv, [pos], oid, mask=msk)
                return cv + hits

            return plsc.parallel_loop(
                0, (nch + NL - 1) // NL, carry=cv_in)(tbody)

        cnt_vec = lax.fori_loop(z0, z1 + 1, zbody, zi)
        cnt = jnp.max(cnt_vec)
        plsc.store_scatter(nbrv, [cnt_vec + iota], zi)

        for i in range(2 * KPROD):
            accv[pl.ds(i * NL, NL)] = zf

        def bbody(jb):
            j16 = jb * NL
            idx = nbrv[pl.ds(j16, NL)]
            gx = plsc.load_gather(pxv, [idx])
            gy = plsc.load_gather(pyv, [idx])
            gz = plsc.load_gather(pzv, [idx])
            dx = gx - qxs
            dy = gy - qys
            dz = gz - qzs
            d2 = jnp.maximum(dx * dx + dy * dy + dz * dz, 1e-20)
            ib = plsc.bitcast(d2, jnp.int32)
            y = plsc.bitcast(jnp.int32(0x5F3759DF) - (ib >> 1), jnp.float32)
            y = y * (1.5 - 0.5 * d2 * y * y)
            y = y * (1.5 - 0.5 * d2 * y * y)
            sq = d2 * y
            relx = dx * inv_rs
            rely = dy * inv_rs
            relz = dz * inv_rs
            norm = sq * inv_rs
            ma = jnp.maximum(jnp.maximum(jnp.abs(relx), jnp.abs(rely)),
                             jnp.abs(relz))
            scale = jnp.where(ma > 1e-8, norm / jnp.maximum(ma, 1e-8), 0.0)
            t0 = jnp.clip(relx * scale + 1.0, 0.0, 2.0)
            t1 = jnp.clip(rely * scale + 1.0, 0.0, 2.0)
            t2 = jnp.clip(relz * scale + 1.0, 0.0, 2.0)
            lo0 = t0.astype(jnp.int32)
            lo1 = t1.astype(jnp.int32)
            lo2 = t2.astype(jnp.int32)
            f0 = t0 - lo0.astype(jnp.float32)
            f1 = t1 - lo1.astype(jnp.float32)
            f2 = t2 - lo2.astype(jnp.float32)
            kb32 = (lo0 * 9 + lo1 * 3 + lo2) * 32
            wb = idx * NL
            validf = ((iota + j16) < cnt_vec).astype(jnp.float32)

            for j in range(NL):
                jj = zi + j
                row = plsc.load_gather(featv, [_take(wb, jj) + iota])
                fa, fb = plsc.unpack(
                    plsc.bitcast(row, jnp.bfloat16),
                    format=plsc.PackFormat.INTERLEAVED)
                f0j = _take(f0, jj)
                f1j = _take(f1, jj)
                f2j = _take(f2, jj)
                aj = _take(validf, jj)
                addr = _take(kb32, jj) + iota
                g0 = aj - f0j * aj
                h0 = f0j * aj
                g1 = 1.0 - f1j
                g2 = 1.0 - f2j
                pgg = g1 * g2
                pfg = f1j * g2
                pgf = g1 * f2j
                pff = f1j * f2j
                for s0, w0 in ((0, g0), (1, h0)):
                    for (s1, s2), p12 in (((0, 0), pgg), ((0, 1), pgf),
                                          ((1, 0), pfg), ((1, 1), pff)):
                        w = w0 * p12
                        o = (s0 * 9 + s1 * 3 + s2) * 32
                        plsc.addupdate_scatter(accv, [addr + o], w * fa)
                        plsc.addupdate_scatter(accv, [addr + (o + 16)],
                                               w * fb)

        nb = (cnt + NL - 1) // NL
        plsc.parallel_loop(0, nb)(bbody)

        @pl.when(m > 0)
        def _():
            pltpu.make_async_copy(
                outv, wsum_h.at[pl.ds((base + m - 1) * 864, 864)],
                sem).wait()

        inv_cnt = 1.0 / jnp.maximum(cnt_vec.astype(jnp.float32), 1.0)
        for i in range(KPROD * 2):
            outv[pl.ds(i * NL, NL)] = accv[pl.ds(i * NL, NL)] * inv_cnt
        pltpu.async_copy(outv, wsum_h.at[pl.ds((base + m) * 864, 864)],
                         sem)
        return carry_

    lax.fori_loop(0, m_per, mbody, 0)
    pltpu.make_async_copy(
        outv, wsum_h.at[pl.ds((base + m_per - 1) * 864, 864)], sem).wait()


def _mm_body(ws_ref, wr_ref, b_ref, o_ref):
    o_ref[...] = (jnp.dot(ws_ref[...], wr_ref[...],
                          preferred_element_type=jnp.float32)
                  + b_ref[0, :][None, :])


def kernel(inp_features, inp_positions, out_positions, extents, kernel, bias):
    n, cin = inp_features.shape
    m = out_positions.shape[0]
    cout = kernel.shape[-1]
    half = cin // 2

    px = inp_positions[:, 0].reshape(n)
    py = inp_positions[:, 1].reshape(n)
    pz = inp_positions[:, 2].reshape(n)
    qx = out_positions[:, 0].reshape(m)
    qy = out_positions[:, 1].reshape(m)
    qz = out_positions[:, 2].reshape(m)
    radii = (0.5 * extents).reshape(m)

    fb = inp_features.astype(jnp.bfloat16)
    lo = lax.bitcast_convert_type(fb[:, :half], jnp.uint16).astype(jnp.uint32)
    hi = lax.bitcast_convert_type(fb[:, half:], jnp.uint16).astype(jnp.uint32)
    featw = lax.bitcast_convert_type(lo | (hi << 16),
                                     jnp.int32).reshape(n * half)

    mesh = plsc.VectorSubcoreMesh(core_axis_name="c", subcore_axis_name="s")
    wsum = pl.kernel(
        _sc_body,
        out_type=jax.ShapeDtypeStruct((m * KPROD * cin,), jnp.float32),
        mesh=mesh,
        scratch_types=[
            pltpu.VMEM((n,), jnp.float32),
            pltpu.VMEM((n,), jnp.float32),
            pltpu.VMEM((n,), jnp.float32),
            pltpu.VMEM((n * half,), jnp.int32),
            pltpu.VMEM((m // 32,), jnp.float32),
            pltpu.VMEM((m // 32,), jnp.float32),
            pltpu.VMEM((m // 32,), jnp.float32),
            pltpu.VMEM((m // 32,), jnp.float32),
            pltpu.VMEM((NBR_CAP,), jnp.int32),
            pltpu.VMEM((ACC_ROWS * 32,), jnp.float32),
            pltpu.VMEM((KPROD * 32,), jnp.float32),
            pltpu.VMEM((n,), jnp.float32),
            pltpu.VMEM((n,), jnp.float32),
            pltpu.VMEM((n,), jnp.float32),
            pltpu.VMEM((n,), jnp.int32),
            pltpu.VMEM((n,), jnp.int32),
            pltpu.VMEM((272,), jnp.int32),
            pltpu.VMEM((272,), jnp.int32),
            pltpu.SemaphoreType.DMA,
        ],
        compiler_params=pltpu.CompilerParams(needs_layout_passes=False),
    )(px, py, pz, qx, qy, qz, radii, featw)

    wr = kernel.reshape(KPROD * cin, cout)
    bias2 = bias.reshape(1, cout)
    out = pl.pallas_call(
        _mm_body,
        out_shape=jax.ShapeDtypeStruct((m, cout), jnp.float32),
    )(wsum.reshape(m, KPROD * cin), wr, bias2)
    return out
